# baseline (speedup 1.0000x reference)
import functools
import math

import jax
import jax.numpy as jnp
from jax import lax
from jax.experimental import pallas as pl
from jax.experimental.pallas import tpu as pltpu
from jax.experimental.pallas import tpu_sc as plsc

_DIM = 2048
_NUM_HEADS = 16
_NC = 2
_NS = 16
_NW = _NC * _NS
_CH = 8
_NB = 2


def _sc_gather(x, table):
    batch, seq = x.shape
    _, d = table.shape
    b = batch * seq
    rows_per_w = b // _NW
    n_chunks = rows_per_w // _CH
    assert n_chunks % (2 * _NB) == 0 and seq % rows_per_w == 0
    w_per_row = seq // rows_per_w
    mesh = plsc.VectorSubcoreMesh(core_axis_name="c", subcore_axis_name="s")

    @functools.partial(
        pl.kernel,
        out_type=jax.ShapeDtypeStruct((b, d), jnp.float32),
        mesh=mesh,
        scratch_types=[
            pltpu.VMEM((rows_per_w,), jnp.int32),
            [pltpu.VMEM((_CH, d), jnp.float32) for _ in range(2 * _NB)],
            [pltpu.SemaphoreType.DMA for _ in range(2 * _NB)],
            [pltpu.SemaphoreType.DMA for _ in range(2 * _NB)],
        ],
    )
    def gather_kernel(x_hbm, table_hbm, out_hbm, idx_v, bufs, gsems, wsems):
        wid = lax.axis_index("s") * _NC + lax.axis_index("c")
        base = wid * rows_per_w
        xr = wid // w_per_row
        xc = (wid % w_per_row) * rows_per_w
        pltpu.sync_copy(x_hbm.at[xr, pl.ds(xc, rows_per_w)], idx_v)

        def start_gather(chunk, bi):
            pltpu.async_copy(
                table_hbm.at[idx_v.at[pl.ds(chunk * _CH, _CH)]], bufs[bi],
                gsems[bi],
            )

        def wait_gather(bi):
            pltpu.make_async_copy(
                table_hbm.at[idx_v.at[pl.ds(0, _CH)]], bufs[bi], gsems[bi]
            ).wait()

        def start_write(chunk, bi):
            pltpu.async_copy(
                bufs[bi], out_hbm.at[pl.ds(base + chunk * _CH, _CH)], wsems[bi]
            )

        def wait_write(bi):
            pltpu.make_async_copy(
                bufs[bi], out_hbm.at[pl.ds(0, _CH)], wsems[bi]
            ).wait()

        for bi in range(_NB):
            start_gather(bi, bi)

        @pl.loop(0, n_chunks, step=2 * _NB)
        def _(j0):
            for bi in range(_NB):
                wait_gather(bi)
            for bi in range(_NB):
                start_write(j0 + bi, bi)

            @pl.when(j0 > 0)
            def _():
                for bi in range(_NB):
                    wait_write(_NB + bi)

            for bi in range(_NB):
                start_gather(j0 + _NB + bi, _NB + bi)
            for bi in range(_NB):
                wait_gather(_NB + bi)
            for bi in range(_NB):
                start_write(j0 + _NB + bi, _NB + bi)

            @pl.when(j0 + 2 * _NB < n_chunks)
            def _():
                for bi in range(_NB):
                    wait_write(bi)
                for bi in range(_NB):
                    start_gather(j0 + 2 * _NB + bi, bi)

        for bi in range(2 * _NB):
            wait_write(bi)

    return gather_kernel(x, table)


_GR = 8


def _sc_gather_spmem(x, table):
    batch, seq = x.shape
    _, d = table.shape
    b = batch * seq
    rows_per_w = b // _NW
    n_groups = rows_per_w // _GR
    assert n_groups % 2 == 0 and seq % rows_per_w == 0
    w_per_row = seq // rows_per_w
    mesh = plsc.VectorSubcoreMesh(core_axis_name="c", subcore_axis_name="s")

    @functools.partial(
        pl.kernel,
        out_type=jax.ShapeDtypeStruct((b, d), jnp.float32),
        mesh=mesh,
        scratch_types=[
            pltpu.VMEM((rows_per_w,), jnp.int32),
            [pltpu.VMEM_SHARED((_NS, _GR, d), jnp.float32) for _ in range(2)],
            [pltpu.SemaphoreType.DMA for _ in range(2)],
            [pltpu.SemaphoreType.DMA for _ in range(2)],
        ],
    )
    def gather_kernel(x_hbm, table_hbm, out_hbm, idx_s, bufs, gsems, wsems):
        sid = lax.axis_index("s")
        wid = sid * _NC + lax.axis_index("c")
        base = wid * rows_per_w
        xr = wid // w_per_row
        xc = (wid % w_per_row) * rows_per_w
        pltpu.sync_copy(x_hbm.at[xr, pl.ds(xc, rows_per_w)], idx_s)

        def start_group(g, bi):
            vals = idx_s[pl.ds(g * _GR, _GR)]
            for r in range(_GR):
                pltpu.async_copy(
                    table_hbm.at[pl.ds(vals[r], 1)],
                    bufs[bi].at[sid, pl.ds(r, 1)], gsems[bi],
                )

        def wait_group(bi):
            pltpu.make_async_copy(
                table_hbm.at[pl.ds(0, _GR)], bufs[bi].at[sid], gsems[bi]
            ).wait()

        def start_write(g, bi):
            pltpu.async_copy(
                bufs[bi].at[sid], out_hbm.at[pl.ds(base + g * _GR, _GR)],
                wsems[bi],
            )

        def wait_write(bi):
            pltpu.make_async_copy(
                bufs[bi].at[sid], out_hbm.at[pl.ds(0, _GR)], wsems[bi]
            ).wait()

        start_group(0, 0)

        @pl.loop(0, n_groups, step=2)
        def _(g0):
            wait_group(0)
            start_write(g0, 0)

            @pl.when(g0 > 0)
            def _():
                wait_write(1)

            start_group(g0 + 1, 1)
            wait_group(1)
            start_write(g0 + 1, 1)

            @pl.when(g0 + 2 < n_groups)
            def _():
                wait_write(0)
                start_group(g0 + 2, 0)

        wait_write(0)
        wait_write(1)

    return gather_kernel(x, table)


def _sc_gather_dual(x, table):
    batch, seq = x.shape
    _, d = table.shape
    b = batch * seq
    rows_per_w = b // _NW
    half_rows = rows_per_w // 2
    n_grp = half_rows // _GR
    assert n_grp % 2 == 0 and seq % rows_per_w == 0
    w_per_row = seq // rows_per_w
    mesh = plsc.VectorSubcoreMesh(core_axis_name="c", subcore_axis_name="s")

    @functools.partial(
        pl.kernel,
        out_type=jax.ShapeDtypeStruct((b, d), jnp.float32),
        mesh=mesh,
        scratch_types=[
            pltpu.VMEM((rows_per_w,), jnp.int32),
            [pltpu.VMEM_SHARED((_NS, _GR, d), jnp.float32) for _ in range(2)],
            [pltpu.VMEM((_GR, d), jnp.float32) for _ in range(2)],
            [pltpu.SemaphoreType.DMA for _ in range(4)],
            [pltpu.SemaphoreType.DMA for _ in range(4)],
        ],
    )
    def gather_kernel(x_hbm, table_hbm, out_hbm, idx_s, abufs, bbufs,
                      gsems, wsems):
        sid = lax.axis_index("s")
        wid = sid * _NC + lax.axis_index("c")
        base = wid * rows_per_w
        xr = wid // w_per_row
        xc = (wid % w_per_row) * rows_per_w
        pltpu.sync_copy(x_hbm.at[xr, pl.ds(xc, rows_per_w)], idx_s)

        def start_a(g, bi):
            vals = idx_s[pl.ds(g * _GR, _GR)]
            for r in range(_GR):
                pltpu.async_copy(
                    table_hbm.at[pl.ds(vals[r], 1)],
                    abufs[bi].at[sid, pl.ds(r, 1)], gsems[bi],
                )

        def wait_a(bi):
            pltpu.make_async_copy(
                table_hbm.at[pl.ds(0, _GR)], abufs[bi].at[sid], gsems[bi]
            ).wait()

        def start_wa(g, bi):
            pltpu.async_copy(
                abufs[bi].at[sid], out_hbm.at[pl.ds(base + g * _GR, _GR)],
                wsems[bi],
            )

        def wait_wa(bi):
            pltpu.make_async_copy(
                abufs[bi].at[sid], out_hbm.at[pl.ds(0, _GR)], wsems[bi]
            ).wait()

        def start_b(g, bi):
            pltpu.async_copy(
                table_hbm.at[idx_s.at[pl.ds(half_rows + g * _GR, _GR)]],
                bbufs[bi], gsems[2 + bi],
            )

        def wait_b(bi):
            pltpu.make_async_copy(
                table_hbm.at[idx_s.at[pl.ds(0, _GR)]], bbufs[bi], gsems[2 + bi]
            ).wait()

        def start_wb(g, bi):
            pltpu.async_copy(
                bbufs[bi],
                out_hbm.at[pl.ds(base + half_rows + g * _GR, _GR)],
                wsems[2 + bi],
            )

        def wait_wb(bi):
            pltpu.make_async_copy(
                bbufs[bi], out_hbm.at[pl.ds(0, _GR)], wsems[2 + bi]
            ).wait()

        start_a(0, 0)
        start_b(0, 0)

        @pl.loop(0, n_grp, step=2)
        def _(g0):
            wait_a(0)
            start_wa(g0, 0)
            wait_b(0)
            start_wb(g0, 0)

            @pl.when(g0 > 0)
            def _():
                wait_wa(1)
                wait_wb(1)

            start_a(g0 + 1, 1)
            start_b(g0 + 1, 1)
            wait_a(1)
            start_wa(g0 + 1, 1)
            wait_b(1)
            start_wb(g0 + 1, 1)

            @pl.when(g0 + 2 < n_grp)
            def _():
                wait_wa(0)
                wait_wb(0)
                start_a(g0 + 2, 0)
                start_b(g0 + 2, 0)

        wait_wa(0)
        wait_wa(1)
        wait_wb(0)
        wait_wb(1)

    return gather_kernel(x, table)


def _tc_body(t_ref, w1_ref, b1_ref, w2_ref, b2_ref, c_ref, pos_ref):
    half = _DIM // 2
    i = lax.broadcasted_iota(jnp.int32, (4, half), 1).astype(jnp.float32)
    freqs = jnp.exp(i * (-math.log(10000.0) / half))
    args = t_ref[...] * freqs
    emb = jnp.concatenate([jnp.sin(args), jnp.cos(args)], axis=-1)
    h = jnp.dot(emb, w1_ref[...], preferred_element_type=jnp.float32)
    h = h + b1_ref[...]
    h = h * (1.0 / (1.0 + jnp.exp(-h)))
    c = jnp.dot(h, w2_ref[...], preferred_element_type=jnp.float32)
    c_ref[...] = c + b2_ref[...]

    head_dim = _DIM // _NUM_HEADS
    hh = head_dim // 2
    ln = pos_ref.shape[1]
    p = lax.broadcasted_iota(jnp.int32, (ln, hh), 0).astype(jnp.float32)
    fi = lax.broadcasted_iota(jnp.int32, (ln, hh), 1).astype(jnp.float32)
    inv_freq = jnp.exp(fi * (-2.0 * math.log(10000.0) / head_dim))
    fr = p * inv_freq
    emb2 = jnp.concatenate([fr, fr], axis=-1)
    pos_ref[0] = jnp.cos(emb2)
    pos_ref[1] = jnp.sin(emb2)


def _tc_mlp_rotary(t, w1, b1, w2, b2, seq_len):
    head_dim = _DIM // _NUM_HEADS
    return pl.pallas_call(
        _tc_body,
        out_shape=(
            jax.ShapeDtypeStruct((4, _DIM), jnp.float32),
            jax.ShapeDtypeStruct((2, seq_len, head_dim), jnp.float32),
        ),
    )(t, w1, b1, w2, b2)


def kernel(x, t, table, W1, b1, W2, b2):
    batch, seq_len = x.shape
    h = _sc_gather_dual(x.astype(jnp.int32), table).reshape(batch, seq_len, _DIM)
    c, pos = _tc_mlp_rotary(
        t.reshape(4, 1), W1, b1.reshape(1, _DIM), W2, b2.reshape(1, _DIM), seq_len
    )
    return (h, c, pos)


# speedup vs baseline: 1.0375x; 1.0375x over previous
import functools
import math

import jax
import jax.numpy as jnp
from jax import lax
from jax.experimental import pallas as pl
from jax.experimental.pallas import tpu as pltpu
from jax.experimental.pallas import tpu_sc as plsc

_DIM = 2048
_NUM_HEADS = 16
_NC = 2
_NS = 16
_NW = _NC * _NS
_CH = 8
_NB = 2


def _sc_gather(x, table):
    batch, seq = x.shape
    _, d = table.shape
    b = batch * seq
    rows_per_w = b // _NW
    n_chunks = rows_per_w // _CH
    assert n_chunks % (2 * _NB) == 0 and seq % rows_per_w == 0
    w_per_row = seq // rows_per_w
    mesh = plsc.VectorSubcoreMesh(core_axis_name="c", subcore_axis_name="s")

    @functools.partial(
        pl.kernel,
        out_type=jax.ShapeDtypeStruct((b, d), jnp.float32),
        mesh=mesh,
        scratch_types=[
            pltpu.VMEM((rows_per_w,), jnp.int32),
            [pltpu.VMEM((_CH, d), jnp.float32) for _ in range(2 * _NB)],
            [pltpu.SemaphoreType.DMA for _ in range(2 * _NB)],
            [pltpu.SemaphoreType.DMA for _ in range(2 * _NB)],
        ],
    )
    def gather_kernel(x_hbm, table_hbm, out_hbm, idx_v, bufs, gsems, wsems):
        wid = lax.axis_index("s") * _NC + lax.axis_index("c")
        base = wid * rows_per_w
        xr = wid // w_per_row
        xc = (wid % w_per_row) * rows_per_w
        pltpu.sync_copy(x_hbm.at[xr, pl.ds(xc, rows_per_w)], idx_v)

        def start_gather(chunk, bi):
            pltpu.async_copy(
                table_hbm.at[idx_v.at[pl.ds(chunk * _CH, _CH)]], bufs[bi],
                gsems[bi],
            )

        def wait_gather(bi):
            pltpu.make_async_copy(
                table_hbm.at[idx_v.at[pl.ds(0, _CH)]], bufs[bi], gsems[bi]
            ).wait()

        def start_write(chunk, bi):
            pltpu.async_copy(
                bufs[bi], out_hbm.at[pl.ds(base + chunk * _CH, _CH)], wsems[bi]
            )

        def wait_write(bi):
            pltpu.make_async_copy(
                bufs[bi], out_hbm.at[pl.ds(0, _CH)], wsems[bi]
            ).wait()

        for bi in range(_NB):
            start_gather(bi, bi)

        @pl.loop(0, n_chunks, step=2 * _NB)
        def _(j0):
            for bi in range(_NB):
                wait_gather(bi)
            for bi in range(_NB):
                start_write(j0 + bi, bi)

            @pl.when(j0 > 0)
            def _():
                for bi in range(_NB):
                    wait_write(_NB + bi)

            for bi in range(_NB):
                start_gather(j0 + _NB + bi, _NB + bi)
            for bi in range(_NB):
                wait_gather(_NB + bi)
            for bi in range(_NB):
                start_write(j0 + _NB + bi, _NB + bi)

            @pl.when(j0 + 2 * _NB < n_chunks)
            def _():
                for bi in range(_NB):
                    wait_write(bi)
                for bi in range(_NB):
                    start_gather(j0 + 2 * _NB + bi, bi)

        for bi in range(2 * _NB):
            wait_write(bi)

    return gather_kernel(x, table)


_GR = 16


def _sc_gather_spmem(x, table):
    batch, seq = x.shape
    _, d = table.shape
    b = batch * seq
    rows_per_w = b // _NW
    n_groups = rows_per_w // _GR
    assert n_groups % 2 == 0 and seq % rows_per_w == 0
    w_per_row = seq // rows_per_w
    mesh = plsc.VectorSubcoreMesh(core_axis_name="c", subcore_axis_name="s")

    @functools.partial(
        pl.kernel,
        out_type=jax.ShapeDtypeStruct((b, d), jnp.float32),
        mesh=mesh,
        scratch_types=[
            pltpu.VMEM((rows_per_w,), jnp.int32),
            [pltpu.VMEM_SHARED((_NS, _GR, d), jnp.float32) for _ in range(2)],
            [pltpu.SemaphoreType.DMA for _ in range(2)],
            [pltpu.SemaphoreType.DMA for _ in range(2)],
        ],
    )
    def gather_kernel(x_hbm, table_hbm, out_hbm, idx_s, bufs, gsems, wsems):
        sid = lax.axis_index("s")
        wid = sid * _NC + lax.axis_index("c")
        base = wid * rows_per_w
        xr = wid // w_per_row
        xc = (wid % w_per_row) * rows_per_w
        pltpu.sync_copy(x_hbm.at[xr, pl.ds(xc, rows_per_w)], idx_s)

        def start_group(g, bi):
            vals = idx_s[pl.ds(g * _GR, _GR)]
            for r in range(_GR):
                pltpu.async_copy(
                    table_hbm.at[pl.ds(vals[r], 1)],
                    bufs[bi].at[sid, pl.ds(r, 1)], gsems[bi],
                )

        def wait_group(bi):
            pltpu.make_async_copy(
                table_hbm.at[pl.ds(0, _GR)], bufs[bi].at[sid], gsems[bi]
            ).wait()

        def start_write(g, bi):
            pltpu.async_copy(
                bufs[bi].at[sid], out_hbm.at[pl.ds(base + g * _GR, _GR)],
                wsems[bi],
            )

        def wait_write(bi):
            pltpu.make_async_copy(
                bufs[bi].at[sid], out_hbm.at[pl.ds(0, _GR)], wsems[bi]
            ).wait()

        start_group(0, 0)

        @pl.loop(0, n_groups, step=2)
        def _(g0):
            @pl.when(g0 > 0)
            def _():
                wait_write(1)

            start_group(g0 + 1, 1)
            wait_group(0)
            start_write(g0, 0)
            wait_group(1)
            start_write(g0 + 1, 1)

            @pl.when(g0 + 2 < n_groups)
            def _():
                wait_write(0)
                start_group(g0 + 2, 0)

        wait_write(0)
        wait_write(1)

    return gather_kernel(x, table)


def _sc_gather_dual(x, table):
    batch, seq = x.shape
    _, d = table.shape
    b = batch * seq
    rows_per_w = b // _NW
    half_rows = rows_per_w // 2
    n_grp = half_rows // _GR
    assert n_grp % 2 == 0 and seq % rows_per_w == 0
    w_per_row = seq // rows_per_w
    mesh = plsc.VectorSubcoreMesh(core_axis_name="c", subcore_axis_name="s")

    @functools.partial(
        pl.kernel,
        out_type=jax.ShapeDtypeStruct((b, d), jnp.float32),
        mesh=mesh,
        scratch_types=[
            pltpu.VMEM((rows_per_w,), jnp.int32),
            [pltpu.VMEM_SHARED((_NS, _GR, d), jnp.float32) for _ in range(2)],
            [pltpu.VMEM((_GR, d), jnp.float32) for _ in range(2)],
            [pltpu.SemaphoreType.DMA for _ in range(4)],
            [pltpu.SemaphoreType.DMA for _ in range(4)],
        ],
    )
    def gather_kernel(x_hbm, table_hbm, out_hbm, idx_s, abufs, bbufs,
                      gsems, wsems):
        sid = lax.axis_index("s")
        wid = sid * _NC + lax.axis_index("c")
        base = wid * rows_per_w
        xr = wid // w_per_row
        xc = (wid % w_per_row) * rows_per_w
        pltpu.sync_copy(x_hbm.at[xr, pl.ds(xc, rows_per_w)], idx_s)

        def start_a(g, bi):
            vals = idx_s[pl.ds(g * _GR, _GR)]
            for r in range(_GR):
                pltpu.async_copy(
                    table_hbm.at[pl.ds(vals[r], 1)],
                    abufs[bi].at[sid, pl.ds(r, 1)], gsems[bi],
                )

        def wait_a(bi):
            pltpu.make_async_copy(
                table_hbm.at[pl.ds(0, _GR)], abufs[bi].at[sid], gsems[bi]
            ).wait()

        def start_wa(g, bi):
            pltpu.async_copy(
                abufs[bi].at[sid], out_hbm.at[pl.ds(base + g * _GR, _GR)],
                wsems[bi],
            )

        def wait_wa(bi):
            pltpu.make_async_copy(
                abufs[bi].at[sid], out_hbm.at[pl.ds(0, _GR)], wsems[bi]
            ).wait()

        def start_b(g, bi):
            pltpu.async_copy(
                table_hbm.at[idx_s.at[pl.ds(half_rows + g * _GR, _GR)]],
                bbufs[bi], gsems[2 + bi],
            )

        def wait_b(bi):
            pltpu.make_async_copy(
                table_hbm.at[idx_s.at[pl.ds(0, _GR)]], bbufs[bi], gsems[2 + bi]
            ).wait()

        def start_wb(g, bi):
            pltpu.async_copy(
                bbufs[bi],
                out_hbm.at[pl.ds(base + half_rows + g * _GR, _GR)],
                wsems[2 + bi],
            )

        def wait_wb(bi):
            pltpu.make_async_copy(
                bbufs[bi], out_hbm.at[pl.ds(0, _GR)], wsems[2 + bi]
            ).wait()

        start_a(0, 0)
        start_b(0, 0)

        @pl.loop(0, n_grp, step=2)
        def _(g0):
            wait_a(0)
            start_wa(g0, 0)
            wait_b(0)
            start_wb(g0, 0)

            @pl.when(g0 > 0)
            def _():
                wait_wa(1)
                wait_wb(1)

            start_a(g0 + 1, 1)
            start_b(g0 + 1, 1)
            wait_a(1)
            start_wa(g0 + 1, 1)
            wait_b(1)
            start_wb(g0 + 1, 1)

            @pl.when(g0 + 2 < n_grp)
            def _():
                wait_wa(0)
                wait_wb(0)
                start_a(g0 + 2, 0)
                start_b(g0 + 2, 0)

        wait_wa(0)
        wait_wa(1)
        wait_wb(0)
        wait_wb(1)

    return gather_kernel(x, table)


def _tc_body(t_ref, w1_ref, b1_ref, w2_ref, b2_ref, c_ref, pos_ref):
    half = _DIM // 2
    i = lax.broadcasted_iota(jnp.int32, (4, half), 1).astype(jnp.float32)
    freqs = jnp.exp(i * (-math.log(10000.0) / half))
    args = t_ref[...] * freqs
    emb = jnp.concatenate([jnp.sin(args), jnp.cos(args)], axis=-1)
    h = jnp.dot(emb, w1_ref[...], preferred_element_type=jnp.float32)
    h = h + b1_ref[...]
    h = h * (1.0 / (1.0 + jnp.exp(-h)))
    c = jnp.dot(h, w2_ref[...], preferred_element_type=jnp.float32)
    c_ref[...] = c + b2_ref[...]

    head_dim = _DIM // _NUM_HEADS
    hh = head_dim // 2
    ln = pos_ref.shape[1]
    p = lax.broadcasted_iota(jnp.int32, (ln, hh), 0).astype(jnp.float32)
    fi = lax.broadcasted_iota(jnp.int32, (ln, hh), 1).astype(jnp.float32)
    inv_freq = jnp.exp(fi * (-2.0 * math.log(10000.0) / head_dim))
    fr = p * inv_freq
    emb2 = jnp.concatenate([fr, fr], axis=-1)
    pos_ref[0] = jnp.cos(emb2)
    pos_ref[1] = jnp.sin(emb2)


def _tc_mlp_rotary(t, w1, b1, w2, b2, seq_len):
    head_dim = _DIM // _NUM_HEADS
    return pl.pallas_call(
        _tc_body,
        out_shape=(
            jax.ShapeDtypeStruct((4, _DIM), jnp.float32),
            jax.ShapeDtypeStruct((2, seq_len, head_dim), jnp.float32),
        ),
    )(t, w1, b1, w2, b2)


def kernel(x, t, table, W1, b1, W2, b2):
    batch, seq_len = x.shape
    h = _sc_gather_spmem(x.astype(jnp.int32), table).reshape(batch, seq_len, _DIM)
    c, pos = _tc_mlp_rotary(
        t.reshape(4, 1), W1, b1.reshape(1, _DIM), W2, b2.reshape(1, _DIM), seq_len
    )
    return (h, c, pos)


# speedup vs baseline: 1.0379x; 1.0004x over previous
import functools
import math

import jax
import jax.numpy as jnp
from jax import lax
from jax.experimental import pallas as pl
from jax.experimental.pallas import tpu as pltpu
from jax.experimental.pallas import tpu_sc as plsc

_DIM = 2048
_NUM_HEADS = 16
_NC = 2
_NS = 16
_NW = _NC * _NS
_GR = 16


def _sc_gather_spmem(x, table):
    batch, seq = x.shape
    _, d = table.shape
    b = batch * seq
    rows_per_w = b // _NW
    n_groups = rows_per_w // _GR
    assert n_groups % 2 == 0 and seq % rows_per_w == 0
    w_per_row = seq // rows_per_w
    mesh = plsc.VectorSubcoreMesh(core_axis_name="c", subcore_axis_name="s")

    @functools.partial(
        pl.kernel,
        out_type=jax.ShapeDtypeStruct((b, d), jnp.float32),
        mesh=mesh,
        scratch_types=[
            pltpu.VMEM((rows_per_w,), jnp.int32),
            [pltpu.VMEM_SHARED((_NS, _GR, d), jnp.float32) for _ in range(2)],
            [pltpu.SemaphoreType.DMA for _ in range(2)],
            [pltpu.SemaphoreType.DMA for _ in range(2)],
        ],
    )
    def gather_kernel(x_hbm, table_hbm, out_hbm, idx_s, bufs, gsems, wsems):
        sid = lax.axis_index("s")
        wid = sid * _NC + lax.axis_index("c")
        base = wid * rows_per_w
        xr = wid // w_per_row
        xc = (wid % w_per_row) * rows_per_w
        pltpu.sync_copy(x_hbm.at[xr, pl.ds(xc, rows_per_w)], idx_s)

        def start_group(g, bi):
            vals = idx_s[pl.ds(g * _GR, _GR)]
            for r in range(_GR):
                pltpu.async_copy(
                    table_hbm.at[pl.ds(vals[r], 1)],
                    bufs[bi].at[sid, pl.ds(r, 1)], gsems[bi],
                )

        def wait_group(bi):
            pltpu.make_async_copy(
                table_hbm.at[pl.ds(0, _GR)], bufs[bi].at[sid], gsems[bi]
            ).wait()

        def start_write(g, bi):
            pltpu.async_copy(
                bufs[bi].at[sid], out_hbm.at[pl.ds(base + g * _GR, _GR)],
                wsems[bi],
            )

        def wait_write(bi):
            pltpu.make_async_copy(
                bufs[bi].at[sid], out_hbm.at[pl.ds(0, _GR)], wsems[bi]
            ).wait()

        start_group(0, 0)

        @pl.loop(0, n_groups, step=2)
        def _(g0):
            @pl.when(g0 > 0)
            def _():
                wait_write(1)

            start_group(g0 + 1, 1)
            wait_group(0)
            start_write(g0, 0)
            wait_group(1)
            start_write(g0 + 1, 1)

            @pl.when(g0 + 2 < n_groups)
            def _():
                wait_write(0)
                start_group(g0 + 2, 0)

        wait_write(0)
        wait_write(1)

    return gather_kernel(x, table)


def _tc_body(t_ref, w1_ref, b1_ref, w2_ref, b2_ref, c_ref, pos_ref):
    half = _DIM // 2
    i = lax.broadcasted_iota(jnp.int32, (4, half), 1).astype(jnp.float32)
    freqs = jnp.exp(i * (-math.log(10000.0) / half))
    args = t_ref[...] * freqs
    emb = jnp.concatenate([jnp.sin(args), jnp.cos(args)], axis=-1)
    h = jnp.dot(emb, w1_ref[...], preferred_element_type=jnp.float32)
    h = h + b1_ref[...]
    h = h * (1.0 / (1.0 + jnp.exp(-h)))
    c = jnp.dot(h, w2_ref[...], preferred_element_type=jnp.float32)
    c_ref[...] = c + b2_ref[...]

    head_dim = _DIM // _NUM_HEADS
    hh = head_dim // 2
    ln = pos_ref.shape[1]
    p = lax.broadcasted_iota(jnp.int32, (ln, hh), 0).astype(jnp.float32)
    fi = lax.broadcasted_iota(jnp.int32, (ln, hh), 1).astype(jnp.float32)
    inv_freq = jnp.exp(fi * (-2.0 * math.log(10000.0) / head_dim))
    fr = p * inv_freq
    emb2 = jnp.concatenate([fr, fr], axis=-1)
    pos_ref[0] = jnp.cos(emb2)
    pos_ref[1] = jnp.sin(emb2)


def _tc_mlp_rotary(t, w1, b1, w2, b2, seq_len):
    head_dim = _DIM // _NUM_HEADS
    return pl.pallas_call(
        _tc_body,
        out_shape=(
            jax.ShapeDtypeStruct((4, _DIM), jnp.float32),
            jax.ShapeDtypeStruct((2, seq_len, head_dim), jnp.float32),
        ),
    )(t, w1, b1, w2, b2)


def kernel(x, t, table, W1, b1, W2, b2):
    batch, seq_len = x.shape
    h = _sc_gather_spmem(x.astype(jnp.int32), table).reshape(batch, seq_len, _DIM)
    c, pos = _tc_mlp_rotary(
        t.reshape(4, 1), W1, b1.reshape(1, _DIM), W2, b2.reshape(1, _DIM), seq_len
    )
    return (h, c, pos)
